# baseline (speedup 1.0000x reference)
import jax
import jax.numpy as jnp
from jax.experimental import pallas as pl
from jax.experimental.pallas import tpu as pltpu
from jax import lax

D_MODEL = 768
NUM_EXPERTS = 8
TOP_K = 2
GROUPS = 16
SUB = 256
TOKENS_PER_BLOCK = GROUPS * SUB
LANES = GROUPS * NUM_EXPERTS


def _butterfly(a, lane, op):
    for s in (4, 2, 1):
        left = jnp.roll(a, -s, axis=1)
        right = jnp.roll(a, s, axis=1)
        partner = jnp.where((lane & s) == 0, left, right)
        a = op(a, partner)
    return a


def _router_block(x_ref, ww_ref, logits_ref, w_ref, idx_ref, sc_ref):
    logits = sc_ref[...]

    lane = lax.broadcasted_iota(jnp.int32, logits.shape, 1)
    sub = lane & (NUM_EXPERTS - 1)
    iotaf = sub.astype(jnp.float32)
    big = jnp.float32(NUM_EXPERTS)

    v1 = _butterfly(logits, lane, jnp.maximum)
    e = jnp.exp(logits - v1)
    z = _butterfly(e, lane, jnp.add)
    i1 = _butterfly(jnp.where(logits == v1, iotaf, big), lane, jnp.minimum)
    masked = jnp.where(iotaf == i1, -jnp.inf, logits)
    v2 = _butterfly(masked, lane, jnp.maximum)
    i2 = _butterfly(jnp.where(masked == v2, iotaf, big), lane, jnp.minimum)

    e2 = jnp.exp(v2 - v1)
    denom = 1.0 + e2 + 1e-09 * z
    w1 = 1.0 / denom
    w2 = e2 / denom

    logits_ref[...] = logits
    w_ref[...] = jnp.where(sub == 0, w1, jnp.where(sub == 1, w2, 0.0))
    idx_ref[...] = jnp.where(sub == 0, i1, jnp.where(sub == 1, i2, 0.0)).astype(jnp.int32)

    acc = jnp.dot(
        x_ref[0 * SUB : 1 * SUB, :], ww_ref[0], preferred_element_type=jnp.float32
    )
    for j in range(1, GROUPS):
        acc = acc + jnp.dot(
            x_ref[j * SUB : (j + 1) * SUB, :], ww_ref[j],
            preferred_element_type=jnp.float32,
        )
    sc_ref[...] = acc


def kernel(x, W):
    b, s, d = x.shape
    n_rows = b * s
    xf = x.reshape(n_rows, d)

    eye = jnp.eye(GROUPS, dtype=W.dtype)
    ww = jnp.einsum("gh,ed->gdhe", eye, W).reshape(GROUPS, d, LANES)

    n_blocks = n_rows // TOKENS_PER_BLOCK
    grid = (n_blocks + 1,)

    last = n_blocks - 1

    logits_w, w_w, idx_w = pl.pallas_call(
        _router_block,
        grid=grid,
        in_specs=[
            pl.BlockSpec((TOKENS_PER_BLOCK, d), lambda i: (jnp.minimum(i, last), 0)),
            pl.BlockSpec((GROUPS, d, LANES), lambda i: (0, 0, 0)),
        ],
        out_specs=[
            pl.BlockSpec((SUB, LANES), lambda i: (jnp.maximum(i - 1, 0), 0)),
            pl.BlockSpec((SUB, LANES), lambda i: (jnp.maximum(i - 1, 0), 0)),
            pl.BlockSpec((SUB, LANES), lambda i: (jnp.maximum(i - 1, 0), 0)),
        ],
        out_shape=[
            jax.ShapeDtypeStruct((n_blocks * SUB, LANES), jnp.float32),
            jax.ShapeDtypeStruct((n_blocks * SUB, LANES), jnp.float32),
            jax.ShapeDtypeStruct((n_blocks * SUB, LANES), jnp.int32),
        ],
        scratch_shapes=[pltpu.VMEM((SUB, LANES), jnp.float32)],
    )(xf, ww)

    def unwide(a):
        a = a.reshape(n_blocks, SUB, GROUPS, NUM_EXPERTS)
        return a.transpose(0, 2, 1, 3).reshape(n_rows, NUM_EXPERTS)

    logits = unwide(logits_w).reshape(b, s, NUM_EXPERTS)
    w = unwide(w_w)[:, :TOP_K].reshape(b, s, TOP_K)
    idx = unwide(idx_w)[:, :TOP_K].reshape(b, s, TOP_K)
    return (idx, w, logits)


# speedup vs baseline: 1.0044x; 1.0044x over previous
import jax
import jax.numpy as jnp
from jax.experimental import pallas as pl
from jax.experimental.pallas import tpu as pltpu
from jax import lax

D_MODEL = 768
NUM_EXPERTS = 8
TOP_K = 2
GROUPS = 16
SUB = 256
TOKENS_PER_BLOCK = GROUPS * SUB
LANES = GROUPS * NUM_EXPERTS


def _butterfly(a, lane, op):
    for s in (4, 2, 1):
        left = jnp.roll(a, -s, axis=1)
        right = jnp.roll(a, s, axis=1)
        partner = jnp.where((lane & s) == 0, left, right)
        a = op(a, partner)
    return a


N_XREFS = 4
SUBS_PER_XREF = GROUPS // N_XREFS


def _router_block(x0_ref, x1_ref, x2_ref, x3_ref, ww_ref,
                  logits_ref, w_ref, idx_ref, sc_ref):
    logits = sc_ref[...]

    lane = lax.broadcasted_iota(jnp.int32, logits.shape, 1)
    sub = lane & (NUM_EXPERTS - 1)
    iotaf = sub.astype(jnp.float32)
    big = jnp.float32(NUM_EXPERTS)

    v1 = _butterfly(logits, lane, jnp.maximum)
    e = jnp.exp(logits - v1)
    z = _butterfly(e, lane, jnp.add)
    i1 = _butterfly(jnp.where(logits == v1, iotaf, big), lane, jnp.minimum)
    masked = jnp.where(iotaf == i1, -jnp.inf, logits)
    v2 = _butterfly(masked, lane, jnp.maximum)
    i2 = _butterfly(jnp.where(masked == v2, iotaf, big), lane, jnp.minimum)

    e2 = jnp.exp(v2 - v1)
    denom = 1.0 + e2 + 1e-09 * z
    w1 = 1.0 / denom
    w2 = e2 / denom

    logits_ref[...] = logits
    w_ref[...] = jnp.where(sub == 0, w1, jnp.where(sub == 1, w2, 0.0))
    idx_ref[...] = jnp.where(sub == 0, i1, jnp.where(sub == 1, i2, 0.0)).astype(jnp.int32)

    xrefs = (x0_ref, x1_ref, x2_ref, x3_ref)
    acc = None
    for j in range(GROUPS):
        xr = xrefs[j // SUBS_PER_XREF]
        jj = j % SUBS_PER_XREF
        part = jnp.dot(
            xr[jj * SUB : (jj + 1) * SUB, :], ww_ref[j],
            preferred_element_type=jnp.float32,
        )
        acc = part if acc is None else acc + part
    sc_ref[...] = acc


def kernel(x, W):
    b, s, d = x.shape
    n_rows = b * s
    xf = x.reshape(n_rows, d)

    eye = jnp.eye(GROUPS, dtype=W.dtype)
    ww = jnp.einsum("gh,ed->gdhe", eye, W).reshape(GROUPS, d, LANES)

    n_blocks = n_rows // TOKENS_PER_BLOCK
    grid = (n_blocks + 1,)

    last = n_blocks - 1

    logits_w, w_w, idx_w = pl.pallas_call(
        _router_block,
        grid=grid,
        in_specs=[
            pl.BlockSpec(
                (TOKENS_PER_BLOCK // N_XREFS, d),
                lambda i, q=q: (jnp.minimum(i, last) * N_XREFS + q, 0),
            )
            for q in range(N_XREFS)
        ] + [
            pl.BlockSpec((GROUPS, d, LANES), lambda i: (0, 0, 0)),
        ],
        out_specs=[
            pl.BlockSpec((SUB, LANES), lambda i: (jnp.maximum(i - 1, 0), 0)),
            pl.BlockSpec((SUB, LANES), lambda i: (jnp.maximum(i - 1, 0), 0)),
            pl.BlockSpec((SUB, LANES), lambda i: (jnp.maximum(i - 1, 0), 0)),
        ],
        out_shape=[
            jax.ShapeDtypeStruct((n_blocks * SUB, LANES), jnp.float32),
            jax.ShapeDtypeStruct((n_blocks * SUB, LANES), jnp.float32),
            jax.ShapeDtypeStruct((n_blocks * SUB, LANES), jnp.int32),
        ],
        scratch_shapes=[pltpu.VMEM((SUB, LANES), jnp.float32)],
    )(xf, xf, xf, xf, ww)

    def unwide(a):
        a = a.reshape(n_blocks, SUB, GROUPS, NUM_EXPERTS)
        return a.transpose(0, 2, 1, 3).reshape(n_rows, NUM_EXPERTS)

    logits = unwide(logits_w).reshape(b, s, NUM_EXPERTS)
    w = unwide(w_w)[:, :TOP_K].reshape(b, s, TOP_K)
    idx = unwide(idx_w)[:, :TOP_K].reshape(b, s, TOP_K)
    return (idx, w, logits)


# speedup vs baseline: 1.2649x; 1.2594x over previous
import jax
import jax.numpy as jnp
from jax.experimental import pallas as pl
from jax.experimental.pallas import tpu as pltpu
from jax import lax

D_MODEL = 768
NUM_EXPERTS = 8
TOP_K = 2
GROUPS = 16
SUB = 256
TOKENS_PER_BLOCK = GROUPS * SUB
LANES = GROUPS * NUM_EXPERTS


def _butterfly(a, lane, op):
    for s in (4, 2, 1):
        left = jnp.roll(a, -s, axis=1)
        right = jnp.roll(a, s, axis=1)
        partner = jnp.where((lane & s) == 0, left, right)
        a = op(a, partner)
    return a


def _router_block(x_ref, wg_ref, logits_ref, packed_ref, sc_ref, ww_ref):
    i = pl.program_id(0)

    @pl.when(i == 0)
    def _build():
        wt = jnp.transpose(wg_ref[...])
        wt_wide = jnp.concatenate([wt] * GROUPS, axis=1)
        grp = lax.broadcasted_iota(jnp.int32, (D_MODEL, LANES), 1) // NUM_EXPERTS
        for j in range(GROUPS):
            ww_ref[j] = jnp.where(grp == j, wt_wide, 0.0)

    logits = sc_ref[...]

    lane = lax.broadcasted_iota(jnp.int32, logits.shape, 1)
    sub = lane & (NUM_EXPERTS - 1)
    iotaf = sub.astype(jnp.float32)
    big = jnp.float32(NUM_EXPERTS)

    v1 = _butterfly(logits, lane, jnp.maximum)
    e = jnp.exp(logits - v1)
    z = _butterfly(e, lane, jnp.add)
    i1 = _butterfly(jnp.where(logits == v1, iotaf, big), lane, jnp.minimum)
    masked = jnp.where(iotaf == i1, -jnp.inf, logits)
    v2 = _butterfly(masked, lane, jnp.maximum)
    i2 = _butterfly(jnp.where(masked == v2, iotaf, big), lane, jnp.minimum)

    e2 = jnp.exp(v2 - v1)
    denom = 1.0 + e2 + 1e-09 * z
    w1 = 1.0 / denom
    w2 = e2 / denom

    logits_ref[...] = logits
    packed_ref[...] = jnp.where(
        sub == 0, w1,
        jnp.where(sub == 1, w2, jnp.where(sub == 2, i1, jnp.where(sub == 3, i2, 0.0))),
    )

    acc = None
    for j in range(GROUPS):
        part = jnp.dot(
            x_ref[j * SUB : (j + 1) * SUB, :], ww_ref[j],
            preferred_element_type=jnp.float32,
        )
        acc = part if acc is None else acc + part
    sc_ref[...] = acc


def kernel(x, W):
    b, s, d = x.shape
    n_rows = b * s
    xf = x.reshape(n_rows, d)

    n_blocks = n_rows // TOKENS_PER_BLOCK
    grid = (n_blocks + 1,)

    last = n_blocks - 1

    logits_w, packed_w = pl.pallas_call(
        _router_block,
        grid=grid,
        in_specs=[
            pl.BlockSpec((TOKENS_PER_BLOCK, d), lambda i: (jnp.minimum(i, last), 0)),
            pl.BlockSpec((NUM_EXPERTS, d), lambda i: (0, 0)),
        ],
        out_specs=[
            pl.BlockSpec((SUB, LANES), lambda i: (jnp.maximum(i - 1, 0), 0)),
            pl.BlockSpec((SUB, LANES), lambda i: (jnp.maximum(i - 1, 0), 0)),
        ],
        out_shape=[
            jax.ShapeDtypeStruct((n_blocks * SUB, LANES), jnp.float32),
            jax.ShapeDtypeStruct((n_blocks * SUB, LANES), jnp.float32),
        ],
        scratch_shapes=[
            pltpu.VMEM((SUB, LANES), jnp.float32),
            pltpu.VMEM((GROUPS, D_MODEL, LANES), jnp.float32),
        ],
    )(xf, W)

    def unwide(a):
        a = a.reshape(n_blocks, SUB, GROUPS, NUM_EXPERTS)
        return a.transpose(0, 2, 1, 3).reshape(n_rows, NUM_EXPERTS)

    logits = unwide(logits_w).reshape(b, s, NUM_EXPERTS)
    packed = unwide(packed_w)
    w = packed[:, :TOP_K].reshape(b, s, TOP_K)
    idx = packed[:, TOP_K : 2 * TOP_K].astype(jnp.int32).reshape(b, s, TOP_K)
    return (idx, w, logits)
